# baseline (speedup 1.0000x reference)
import functools

import jax
import jax.numpy as jnp
from jax import lax
from jax.experimental import pallas as pl
from jax.experimental.pallas import tpu as pltpu
from jax.experimental.pallas import tpu_sc as plsc

B = 16
N = 100000
NBINS = 512
W = 2048
NWF = N // W
TAIL = N - NWF * W
TVREGS = TAIL // 16
TROWS = TAIL // 128


def _zero_bins(ref):
  z = jnp.zeros((16,), jnp.int32)

  def body(i, _):
    ref[pl.ds(i * 16, 16)] = z
    return 0

  lax.fori_loop(0, NBINS // 16, body, 0)


def _excl_scan(hist_ref, base_ref, offset):
  def body(i, carry):
    h = hist_ref[pl.ds(i * 16, 16)]
    c = plsc.cumsum(h)
    base_ref[pl.ds(i * 16, 16)] = c - h + jnp.broadcast_to(carry, (16,))
    return carry + jnp.sum(h)

  lax.fori_loop(0, NBINS // 16, body, offset)


def _make_kern():
  mesh = plsc.VectorSubcoreMesh(core_axis_name="c", subcore_axis_name="s")
  flat = jax.ShapeDtypeStruct((B * N,), jnp.int32)
  out_type = (flat,) * 6

  scratch = [
      pltpu.VMEM((W * 3,), jnp.float32),
      pltpu.VMEM((W,), jnp.int32),
      pltpu.VMEM((W,), jnp.int32),
      pltpu.VMEM((16, 128), jnp.int32),
      pltpu.VMEM((16, 128), jnp.int32),
      pltpu.VMEM((16, 128), jnp.int32),
      pltpu.VMEM((32,), jnp.int32),
      pltpu.VMEM((32,), jnp.int32),
      pltpu.VMEM((32,), jnp.int32),
      pltpu.VMEM((NBINS,), jnp.int32),
      pltpu.VMEM((NBINS,), jnp.int32),
      pltpu.VMEM((NBINS,), jnp.int32),
      pltpu.VMEM((NBINS,), jnp.int32),
      pltpu.SemaphoreType.DMA,
  ]

  @functools.partial(
      pl.kernel, out_type=out_type, mesh=mesh, scratch_types=scratch,
      compiler_params=pltpu.CompilerParams(needs_layout_passes=False))
  def kern(pc_hbm, keys_hbm, order_hbm, sk_hbm, vlin_hbm, va_hbm, ia_hbm,
           raw, vbuf, kbuf, pos2d, dat_a, dat_b, pos_t, dat_ta, dat_tb,
           hist0, hist1, cnt0, cnt1, sem):
    c = lax.axis_index("c")
    s = lax.axis_index("s")
    ncores = plsc.get_sparse_core_info().num_cores
    reps = -(-B // ncores)

    def run_batch(b):
      bN = b * N
      lanes = lax.iota(jnp.int32, 16)

      _zero_bins(hist0)
      _zero_bins(hist1)

      def p1_vreg(i, _):
        p3 = (i * 16 + lanes) * 3
        x = plsc.load_gather(raw, [p3])
        y = plsc.load_gather(raw, [p3 + 1])
        z = plsc.load_gather(raw, [p3 + 2])
        ix = (x * 63.0).astype(jnp.int32)
        iy = (y * 63.0).astype(jnp.int32)
        iz = (z * 63.0).astype(jnp.int32)
        v = ix * 4096 + iy * 64 + iz
        key = ix * 10000 + iy * 100 + iz
        vbuf[pl.ds(i * 16, 16)] = v
        kbuf[pl.ds(i * 16, 16)] = key
        d0 = v & 511
        c0, l0 = plsc.scan_count(d0)
        plsc.addupdate_scatter(hist0, [d0], c0, mask=l0)
        d1 = v >> 9
        c1, l1 = plsc.scan_count(d1)
        plsc.addupdate_scatter(hist1, [d1], c1, mask=l1)
        return 0

      def p1_window(start, nv, we):
        pltpu.async_copy(
            pc_hbm.at[pl.ds((bN + start) * 3, we * 3)],
            raw.at[pl.ds(0, we * 3)], sem).wait()
        lax.fori_loop(0, nv, p1_vreg, 0)
        o1 = pltpu.async_copy(
            vbuf.at[pl.ds(0, we)], vlin_hbm.at[pl.ds(bN + start, we)], sem)
        o2 = pltpu.async_copy(
            kbuf.at[pl.ds(0, we)], keys_hbm.at[pl.ds(bN + start, we)], sem)
        o1.wait()
        o2.wait()

      def w1(w, _):
        p1_window(w * W, W // 16, W)
        return 0

      lax.fori_loop(0, NWF, w1, 0)
      p1_window(NWF * W, TVREGS, TAIL)

      _excl_scan(hist0, cnt0, bN)
      _excl_scan(hist1, cnt1, bN)

      def rank_pass(in1_hbm, in2_hbm, out1_hbm, out2_hbm, cnt, shift_mode):

        def rank_vreg(start, i, pref, pidx, daref, didx_a, dbref, didx_b):
          v = vbuf[pl.ds(i * 16, 16)]
          if shift_mode == 0:
            d = v & 511
          else:
            d = v >> 9
          cc, ll = plsc.scan_count(d)
          base = plsc.load_gather(cnt, [d])
          pos = base + cc - 1
          plsc.addupdate_scatter(cnt, [d], cc, mask=ll)
          if shift_mode == 0:
            a_val = v
            b_val = start + i * 16 + lanes
          else:
            a_val = kbuf[pl.ds(i * 16, 16)]
            ix = v >> 12
            iy = (v >> 6) & 63
            iz = v & 63
            b_val = ix * 10000 + iy * 100 + iz
          pref[pidx] = pos
          daref[didx_a] = a_val
          dbref[didx_b] = b_val

        def rank_window(start, full):
          we = W if full else TAIL
          i1 = pltpu.async_copy(
              in1_hbm.at[pl.ds(bN + start, we)], vbuf.at[pl.ds(0, we)], sem)
          if shift_mode == 1:
            i2 = pltpu.async_copy(
                in2_hbm.at[pl.ds(bN + start, we)], kbuf.at[pl.ds(0, we)], sem)
            i2.wait()
          i1.wait()

          nv2d = (W // 16) if full else (TROWS * 8)

          def body(i, _):
            r = i // 8
            col = (i % 8) * 16
            idx2 = (r, pl.ds(col, 16))
            rank_vreg(start, i, pos2d, idx2, dat_a, idx2, dat_b, idx2)
            return 0

          lax.fori_loop(0, nv2d, body, 0)
          if not full:
            for j in range(2):
              sl = pl.ds(j * 16, 16)
              rank_vreg(start, TROWS * 8 + j, pos_t, sl, dat_ta, sl,
                        dat_tb, sl)

          nrows = 16 if full else TROWS

          def fire_row(j, _):
            pltpu.async_copy(dat_a.at[j], out1_hbm.at[pos2d.at[j]], sem)
            pltpu.async_copy(dat_b.at[j], out2_hbm.at[pos2d.at[j]], sem)
            return 0

          lax.fori_loop(0, nrows, fire_row, 0)
          if not full:
            pltpu.async_copy(dat_ta, out1_hbm.at[pos_t], sem)
            pltpu.async_copy(dat_tb, out2_hbm.at[pos_t], sem)
          ne = nrows * 128
          pltpu.make_async_copy(
              out1_hbm.at[pl.ds(0, ne)], vbuf.at[pl.ds(0, ne)], sem).wait()
          pltpu.make_async_copy(
              out1_hbm.at[pl.ds(0, ne)], vbuf.at[pl.ds(0, ne)], sem).wait()
          if not full:
            pltpu.make_async_copy(
                out1_hbm.at[pl.ds(0, 32)], dat_ta, sem).wait()
            pltpu.make_async_copy(
                out1_hbm.at[pl.ds(0, 32)], dat_tb, sem).wait()

        def wf(w, _):
          rank_window(w * W, True)
          return 0

        lax.fori_loop(0, NWF, wf, 0)
        rank_window(NWF * W, False)

      rank_pass(vlin_hbm, None, va_hbm, ia_hbm, cnt0, 0)

      def settle(w, _):
        pltpu.async_copy(
            va_hbm.at[pl.ds(bN + w * W, W)], vbuf.at[pl.ds(0, W)], sem).wait()
        return 0

      lax.fori_loop(0, NWF, settle, 0)

      rank_pass(va_hbm, ia_hbm, order_hbm, sk_hbm, cnt1, 1)

    @pl.when(s == 0)
    def _():
      def rep_body(r, _):
        bb = c * reps + r

        @pl.when(bb < B)
        def _():
          run_batch(bb)

        return 0

      lax.fori_loop(0, reps, rep_body, 0)

  return kern


_KERN = None


def kernel(point_cloud):
  global _KERN
  if _KERN is None:
    _KERN = _make_kern()
  pc_flat = point_cloud.reshape(-1)
  keys, order, sk, _, _, _ = _KERN(pc_flat)
  return (keys.reshape(B, N), order.reshape(B, N), sk.reshape(B, N))
